# baseline (speedup 1.0000x reference)
import functools

import jax
import jax.numpy as jnp
import numpy as np
from jax import lax
from jax.experimental import pallas as pl
from jax.experimental.pallas import tpu as pltpu
from jax.experimental.pallas import tpu_sc as plsc

_NBUF = 4


def _pos_encoding(max_seq_len, embed_dim, n=10000.0):
    position = jnp.arange(max_seq_len, dtype=jnp.float32)[:, None]
    division_term = jnp.exp(
        jnp.arange(0, embed_dim, 2, dtype=jnp.float32) * (-np.log(n) / embed_dim)
    )
    pe = jnp.zeros((max_seq_len, embed_dim), dtype=jnp.float32)
    pe = pe.at[:, 0::2].set(jnp.sin(position * division_term))
    pe = pe.at[:, 1::2].set(jnp.cos(position * division_term))
    return pe


@functools.cache
def _make_emb_kernel(B, S, D):
    info = plsc.get_sparse_core_info()
    NC, NS = info.num_cores, info.num_subcores
    NW = NC * NS
    assert B % NW == 0
    b_per_w = B // NW
    H = S // 2
    NB = _NBUF

    mesh = plsc.VectorSubcoreMesh(core_axis_name="c", subcore_axis_name="s")

    @functools.partial(
        pl.kernel,
        out_type=jax.ShapeDtypeStruct((B, S, D), jnp.float32),
        mesh=mesh,
        scratch_types=[
            pltpu.VMEM_SHARED((S, D), jnp.float32),
            [pltpu.VMEM((2, H), jnp.int32) for _ in range(NB)],
            [pltpu.VMEM((S, D), jnp.float32) for _ in range(NB)],
            [pltpu.SemaphoreType.DMA for _ in range(NB)],
            [pltpu.SemaphoreType.DMA for _ in range(NB)],
            [pltpu.SemaphoreType.DMA for _ in range(NB)],
            [pltpu.SemaphoreType.DMA for _ in range(NB)],
            [pltpu.SemaphoreType.DMA for _ in range(NB)],
        ],
    )
    def emb_kernel(
        x_hbm, table_hbm, pos_hbm, out_hbm, pos_sh, idx_v, buf,
        isem, nsem, gsem, g2sem, osem
    ):
        sid = lax.axis_index("s")
        wid = sid * NC + lax.axis_index("c")
        base = wid * b_per_w

        @pl.when(sid == 0)
        def _():
            pltpu.sync_copy(pos_hbm, pos_sh)

        plsc.subcore_barrier()

        def start_prefetch(i):
            s = i % NB
            idx_d = pltpu.async_copy(x_hbm.at[base + i], idx_v[s], isem[s])
            init_d = pltpu.async_copy(pos_sh, buf[s], nsem[s])
            return idx_d, init_d

        def start_gather(i):
            s = i % NB
            g0 = pltpu.async_copy(
                table_hbm.at[idx_v[s].at[0]], buf[s].at[pl.ds(0, H)], gsem[s], add=True
            )
            g1 = pltpu.async_copy(
                table_hbm.at[idx_v[s].at[1]], buf[s].at[pl.ds(H, H)], g2sem[s], add=True
            )
            return g0, g1

        pre = {}
        gat = {}
        out = {}

        pre[0] = start_prefetch(0)
        pre[1] = start_prefetch(1)
        pre[0][0].wait()
        pre[0][1].wait()
        gat[0] = start_gather(0)

        for i in range(b_per_w):
            if i >= 2:
                out[i - 2][0].wait()
                out[i - 2][1].wait()
            if i + 2 < b_per_w:
                pre[i + 2] = start_prefetch(i + 2)
            if i + 1 < b_per_w:
                pre[i + 1][0].wait()
                pre[i + 1][1].wait()
                gat[i + 1] = start_gather(i + 1)
            s = i % NB
            W0 = (H // 8) * 8
            gat[i][0].wait()
            o0 = pltpu.async_copy(
                buf[s].at[pl.ds(0, W0)], out_hbm.at[base + i].at[pl.ds(0, W0)], osem[s]
            )
            gat[i][1].wait()
            o1 = pltpu.async_copy(
                buf[s].at[pl.ds(W0, S - W0)],
                out_hbm.at[base + i].at[pl.ds(W0, S - W0)],
                osem[s],
            )
            out[i] = (o0, o1)

        for j in (b_per_w - 2, b_per_w - 1):
            out[j][0].wait()
            out[j][1].wait()

    return emb_kernel


def kernel(x, embedding_weight):
    B, S = x.shape
    D = embedding_weight.shape[1]
    pos = _pos_encoding(S, D)
    x3 = x.astype(jnp.int32).reshape(B, 2, S // 2)
    return _make_emb_kernel(B, S, D)(x3, embedding_weight, pos)


# speedup vs baseline: 1.0189x; 1.0189x over previous
import functools

import jax
import jax.numpy as jnp
import numpy as np
from jax import lax
from jax.experimental import pallas as pl
from jax.experimental.pallas import tpu as pltpu
from jax.experimental.pallas import tpu_sc as plsc

_NBUF = 4


def _pos_encoding(max_seq_len, embed_dim, n=10000.0):
    position = jnp.arange(max_seq_len, dtype=jnp.float32)[:, None]
    division_term = jnp.exp(
        jnp.arange(0, embed_dim, 2, dtype=jnp.float32) * (-np.log(n) / embed_dim)
    )
    pe = jnp.zeros((max_seq_len, embed_dim), dtype=jnp.float32)
    pe = pe.at[:, 0::2].set(jnp.sin(position * division_term))
    pe = pe.at[:, 1::2].set(jnp.cos(position * division_term))
    return pe


@functools.cache
def _make_emb_kernel(B, S, D):
    info = plsc.get_sparse_core_info()
    NC, NS = info.num_cores, info.num_subcores
    NW = NC * NS
    assert B % NW == 0
    b_per_w = B // NW
    H = S // 2
    NB = _NBUF

    mesh = plsc.VectorSubcoreMesh(core_axis_name="c", subcore_axis_name="s")

    @functools.partial(
        pl.kernel,
        out_type=jax.ShapeDtypeStruct((B, S, D), jnp.float32),
        mesh=mesh,
        scratch_types=[
            pltpu.VMEM_SHARED((S, D), jnp.float32),
            pltpu.VMEM((2 * b_per_w, H), jnp.int32),
            [pltpu.VMEM((S, D), jnp.float32) for _ in range(NB)],
            pltpu.SemaphoreType.DMA,
            [pltpu.SemaphoreType.DMA for _ in range(NB)],
            [pltpu.SemaphoreType.DMA for _ in range(NB)],
            [pltpu.SemaphoreType.DMA for _ in range(NB)],
        ],
    )
    def emb_kernel(
        x_hbm, table_hbm, pos_hbm, out_hbm, pos_sh, idx_v, buf, isem, nsem, gsem, osem
    ):
        sid = lax.axis_index("s")
        wid = sid * NC + lax.axis_index("c")
        base = wid * b_per_w

        @pl.when(sid == 0)
        def _():
            pltpu.sync_copy(pos_hbm, pos_sh)

        plsc.subcore_barrier()

        idx_d = pltpu.async_copy(
            x_hbm.at[pl.ds(2 * base, 2 * b_per_w)], idx_v, isem
        )

        def start_init(i):
            s = i % NB
            return pltpu.async_copy(pos_sh, buf[s], nsem[s])

        def start_gather(i):
            s = i % NB
            g0 = pltpu.async_copy(
                table_hbm.at[idx_v.at[2 * i]], buf[s].at[pl.ds(0, H)], gsem[s],
                add=True,
            )
            g1 = pltpu.async_copy(
                table_hbm.at[idx_v.at[2 * i + 1]], buf[s].at[pl.ds(H, H)], gsem[s],
                add=True,
            )
            return g0, g1

        ini = {}
        gat = {}
        out = {}

        ini[0] = start_init(0)
        ini[1] = start_init(1)
        idx_d.wait()
        ini[0].wait()
        gat[0] = start_gather(0)

        for i in range(b_per_w):
            if i + 1 < b_per_w:
                ini[i + 1].wait()
                gat[i + 1] = start_gather(i + 1)
            if i >= 2:
                out[i - 2].wait()
            if i + 2 < b_per_w:
                ini[i + 2] = start_init(i + 2)
            gat[i][0].wait()
            gat[i][1].wait()
            s = i % NB
            out[i] = pltpu.async_copy(buf[s], out_hbm.at[base + i], osem[s])

        out[b_per_w - 2].wait()
        out[b_per_w - 1].wait()

    return emb_kernel


def kernel(x, embedding_weight):
    B, S = x.shape
    D = embedding_weight.shape[1]
    pos = _pos_encoding(S, D)
    x2 = x.astype(jnp.int32).reshape(2 * B, S // 2)
    return _make_emb_kernel(B, S, D)(x2, embedding_weight, pos)
